# baseline (speedup 1.0000x reference)
import functools

import jax
import jax.numpy as jnp
from jax import lax
from jax.experimental import pallas as pl
from jax.experimental.pallas import tpu as pltpu
from jax.experimental.pallas import tpu_sc as plsc

_K = 256
_ROWS = 64
_COLS = 8192
_NC = 2
_NS = 16
_NW = _NC * _NS
_NV = _COLS // 16
_UNROLL = 8
_SC_ROWS = 32
_RPW = _SC_ROWS // _NW


def _u32(val):
    return jnp.full((16,), val, jnp.uint32)


def _sc_body(x_hbm, o_hbm, xv, kv, hist, *sems):
    wid = lax.axis_index("s") * _NC + lax.axis_index("c")
    base = wid * _RPW
    in_sems = sems[:_RPW]
    out_sems = sems[_RPW:]
    in_descs = [
        pltpu.async_copy(x_hbm.at[base + r],
                         xv.at[pl.ds(r * _COLS, _COLS)], in_sems[r])
        for r in range(_RPW)
    ]
    out_descs = []

    zeros16 = jnp.zeros((16,), jnp.int32)
    ones16 = jnp.ones((16,), jnp.int32)
    iota16 = lax.iota(jnp.int32, 16)
    msb = _u32(0x80000000)

    for r in range(_RPW):
        off = r * _COLS
        xrow = xv.at[pl.ds(off, _COLS)]
        krow = kv.at[pl.ds(off, _COLS)]
        in_descs[r].wait()

        prefix = _u32(0)
        kk = jnp.int32(_K)

        for rnd, shift in enumerate((24, 16, 8, 0)):
            shv = _u32(shift)
            pshv = _u32(shift + 8)
            ff = _u32(0xFF)

            if rnd == 0:
                for i in range(256):
                    hist[pl.ds(i * 16, 16)] = zeros16
                lane_off = iota16 * 256

                @plsc.parallel_loop(0, _NV, unroll=_UNROLL)
                def hist_body(i, _shv=shv):
                    v = xrow[pl.ds(i * 16, 16)]
                    b = lax.bitcast_convert_type(v, jnp.uint32)
                    u = jnp.where((b & msb) > _u32(0), ~b, b | msb)
                    krow[pl.ds(i * 16, 16)] = u
                    bin_ = lax.convert_element_type(u >> _shv, jnp.int32)
                    plsc.addupdate_scatter(hist, [lane_off + bin_], ones16)

                vecs = []
                for j in range(16):
                    acc16 = hist[pl.ds(j * 16, 16)]
                    for l in range(1, 16):
                        acc16 = acc16 + hist[pl.ds(l * 256 + j * 16, 16)]
                    vecs.append(acc16)
            else:
                for i in range(16):
                    hist[pl.ds(i * 16, 16)] = zeros16

                @plsc.parallel_loop(0, _NV, unroll=_UNROLL)
                def hist_body(i, _shv=shv, _pshv=pshv, _ff=ff,
                              _prefix=prefix):
                    u = krow[pl.ds(i * 16, 16)]
                    m = (u >> _pshv) == _prefix
                    bin_ = lax.convert_element_type((u >> _shv) & _ff,
                                                    jnp.int32)
                    plsc.addupdate_scatter(hist, [bin_], ones16, mask=m)

                vecs = [hist[pl.ds(i * 16, 16)] for i in range(16)]

            sums = [jnp.sum(v) for v in vecs]
            suf = [None] * 16
            acc = jnp.int32(0)
            for j in range(15, -1, -1):
                acc = acc + sums[j]
                suf[j] = acc
            njs = jnp.int32(0)
            for j in range(16):
                njs = njs + jnp.where(suf[j] >= kk, 1, 0)
            jstar = njs - 1

            vstar = zeros16
            suf_star = jnp.int32(0)
            sum_star = jnp.int32(0)
            for j in range(16):
                is_j = jstar == j
                vstar = jnp.where(is_j, vecs[j], vstar)
                suf_star = jnp.where(is_j, suf[j], suf_star)
                sum_star = jnp.where(is_j, sums[j], sum_star)
            above = suf_star - sum_star

            b_suf = lax.rev(plsc.cumsum(lax.rev(vstar, (0,))), (0,))
            cnt_ge = b_suf + above
            bstar = jnp.sum(jnp.where(cnt_ge >= kk, 1, 0)) - 1
            eq = iota16 == bstar
            suf_at = jnp.sum(jnp.where(eq, b_suf, 0))
            hist_at = jnp.sum(jnp.where(eq, vstar, 0))
            kk = kk - (above + suf_at - hist_at)

            digit = lax.convert_element_type(
                jnp.broadcast_to(jstar * 16 + bstar, (16,)), jnp.uint32)
            prefix = (prefix << _u32(8)) | digit

        bbits = jnp.where((prefix & msb) > _u32(0), prefix ^ msb, ~prefix)
        thr = lax.bitcast_convert_type(bbits, jnp.float32)

        @plsc.parallel_loop(0, _NV, unroll=_UNROLL)
        def mask_body(i, _thr=thr):
            v = xrow[pl.ds(i * 16, 16)]
            xrow[pl.ds(i * 16, 16)] = jnp.where(v >= _thr, v, 0.0)

        out_descs.append(
            pltpu.async_copy(xv.at[pl.ds(off, _COLS)],
                             o_hbm.at[base + r], out_sems[r]))

    for d in out_descs:
        d.wait()


_sc_call = functools.partial(
    pl.kernel,
    out_type=jax.ShapeDtypeStruct((_SC_ROWS, _COLS), jnp.float32),
    mesh=plsc.VectorSubcoreMesh(
        core_axis_name="c", subcore_axis_name="s",
        num_cores=_NC, num_subcores=_NS),
    scratch_types=(
        [pltpu.VMEM((_RPW * _COLS,), jnp.float32),
         pltpu.VMEM((_RPW * _COLS,), jnp.uint32),
         pltpu.VMEM((16 * 256,), jnp.int32)]
        + [pltpu.SemaphoreType.DMA] * (2 * _RPW)
    ),
    compiler_params=pltpu.CompilerParams(needs_layout_passes=False),
)(_sc_body)


def _search(key):
    cnt = jnp.sum((key >= 0).astype(jnp.int32), axis=1, keepdims=True)
    t = jnp.where(cnt >= _K, jnp.int32(0), jnp.int32(-2147483648))
    for bit in range(30, -1, -1):
        cand = t | (jnp.int32(1) << bit)
        cnt = jnp.sum((key >= cand).astype(jnp.int32), axis=1, keepdims=True)
        t = jnp.where(cnt >= _K, cand, t)
    return t


def _tc_kernel_body(x_ref, o_ref):
    x = x_ref[...]
    b = jax.lax.bitcast_convert_type(x, jnp.int32)
    key = b ^ ((b >> 31) & jnp.int32(0x7FFFFFFF))

    half = x.shape[0] // 2
    t0 = _search(key[:half])
    t1 = _search(key[half:])
    t = jnp.concatenate([t0, t1], axis=0)

    thr_bits = t ^ ((t >> 31) & jnp.int32(0x7FFFFFFF))
    thr = jax.lax.bitcast_convert_type(thr_bits, jnp.float32)
    o_ref[...] = jnp.where(x >= thr, x, jnp.zeros_like(x))


def _tc_call(x):
    n_tc = _ROWS - _SC_ROWS
    return pl.pallas_call(
        _tc_kernel_body,
        grid=(1,),
        in_specs=[pl.BlockSpec((n_tc, _COLS), lambda i: (1, 0))],
        out_specs=pl.BlockSpec((n_tc, _COLS), lambda i: (0, 0)),
        out_shape=jax.ShapeDtypeStruct((n_tc, _COLS), x.dtype),
    )(x)


@jax.jit
def kernel(x):
    out_sc = _sc_call(x)
    out_tc = _tc_call(x)
    return jnp.concatenate([out_sc, out_tc], axis=0)


# speedup vs baseline: 1.0653x; 1.0653x over previous
import functools

import jax
import jax.numpy as jnp
from jax import lax
from jax.experimental import pallas as pl
from jax.experimental.pallas import tpu as pltpu
from jax.experimental.pallas import tpu_sc as plsc

_K = 256
_ROWS = 64
_COLS = 8192
_NC = 2
_NS = 16
_NW = _NC * _NS
_NV = _COLS // 16
_UNROLL = 8
_SC_ROWS = 32
_RPW = _SC_ROWS // _NW


def _u32(val):
    return jnp.full((16,), val, jnp.uint32)


def _sc_body(x_hbm, o_hbm, xv, kv, hist, *sems):
    wid = lax.axis_index("s") * _NC + lax.axis_index("c")
    base = wid * _RPW
    in_sems = sems[:_RPW]
    out_sems = sems[_RPW:]
    in_descs = [
        pltpu.async_copy(x_hbm.at[base + r],
                         xv.at[pl.ds(r * _COLS, _COLS)], in_sems[r])
        for r in range(_RPW)
    ]
    out_descs = []

    zeros16 = jnp.zeros((16,), jnp.int32)
    ones16 = jnp.ones((16,), jnp.int32)
    iota16 = lax.iota(jnp.int32, 16)
    msb = _u32(0x80000000)

    for r in range(_RPW):
        off = r * _COLS
        xrow = xv.at[pl.ds(off, _COLS)]
        krow = kv.at[pl.ds(off, _COLS)]
        in_descs[r].wait()

        prefix = _u32(0)
        kk = jnp.int32(_K)

        for rnd, shift in enumerate((24, 16, 8, 0)):
            shv = _u32(shift)
            pshv = _u32(shift + 8)
            ff = _u32(0xFF)

            for i in range(16):
                hist[pl.ds(i * 16, 16)] = zeros16

            if rnd == 0:
                @plsc.parallel_loop(0, _NV, unroll=_UNROLL)
                def hist_body(i, _shv=shv):
                    v = xrow[pl.ds(i * 16, 16)]
                    b = lax.bitcast_convert_type(v, jnp.uint32)
                    u = jnp.where((b & msb) > _u32(0), ~b, b | msb)
                    krow[pl.ds(i * 16, 16)] = u
                    bin_ = lax.convert_element_type(u >> _shv, jnp.int32)
                    plsc.addupdate_scatter(hist, [bin_], ones16)
            else:
                @plsc.parallel_loop(0, _NV, unroll=_UNROLL)
                def hist_body(i, _shv=shv, _pshv=pshv, _ff=ff,
                              _prefix=prefix):
                    u = krow[pl.ds(i * 16, 16)]
                    m = (u >> _pshv) == _prefix
                    bin_ = lax.convert_element_type((u >> _shv) & _ff,
                                                    jnp.int32)
                    plsc.addupdate_scatter(hist, [bin_], ones16, mask=m)

            vecs = [hist[pl.ds(i * 16, 16)] for i in range(16)]

            sums = [jnp.sum(v) for v in vecs]
            suf = [None] * 16
            acc = jnp.int32(0)
            for j in range(15, -1, -1):
                acc = acc + sums[j]
                suf[j] = acc
            njs = jnp.int32(0)
            for j in range(16):
                njs = njs + jnp.where(suf[j] >= kk, 1, 0)
            jstar = njs - 1

            vstar = zeros16
            suf_star = jnp.int32(0)
            sum_star = jnp.int32(0)
            for j in range(16):
                is_j = jstar == j
                vstar = jnp.where(is_j, vecs[j], vstar)
                suf_star = jnp.where(is_j, suf[j], suf_star)
                sum_star = jnp.where(is_j, sums[j], sum_star)
            above = suf_star - sum_star

            b_suf = lax.rev(plsc.cumsum(lax.rev(vstar, (0,))), (0,))
            cnt_ge = b_suf + above
            bstar = jnp.sum(jnp.where(cnt_ge >= kk, 1, 0)) - 1
            eq = iota16 == bstar
            suf_at = jnp.sum(jnp.where(eq, b_suf, 0))
            hist_at = jnp.sum(jnp.where(eq, vstar, 0))
            kk = kk - (above + suf_at - hist_at)

            digit = lax.convert_element_type(
                jnp.broadcast_to(jstar * 16 + bstar, (16,)), jnp.uint32)
            prefix = (prefix << _u32(8)) | digit

        bbits = jnp.where((prefix & msb) > _u32(0), prefix ^ msb, ~prefix)
        thr = lax.bitcast_convert_type(bbits, jnp.float32)

        @plsc.parallel_loop(0, _NV, unroll=_UNROLL)
        def mask_body(i, _thr=thr):
            v = xrow[pl.ds(i * 16, 16)]
            xrow[pl.ds(i * 16, 16)] = jnp.where(v >= _thr, v, 0.0)

        out_descs.append(
            pltpu.async_copy(xv.at[pl.ds(off, _COLS)],
                             o_hbm.at[base + r], out_sems[r]))

    for d in out_descs:
        d.wait()


_sc_call = functools.partial(
    pl.kernel,
    out_type=jax.ShapeDtypeStruct((_SC_ROWS, _COLS), jnp.float32),
    mesh=plsc.VectorSubcoreMesh(
        core_axis_name="c", subcore_axis_name="s",
        num_cores=_NC, num_subcores=_NS),
    scratch_types=(
        [pltpu.VMEM((_RPW * _COLS,), jnp.float32),
         pltpu.VMEM((_RPW * _COLS,), jnp.uint32),
         pltpu.VMEM((256,), jnp.int32)]
        + [pltpu.SemaphoreType.DMA] * (2 * _RPW)
    ),
    compiler_params=pltpu.CompilerParams(needs_layout_passes=False),
)(_sc_body)


def _search(key):
    cnt = jnp.sum((key >= 0).astype(jnp.int32), axis=1, keepdims=True)
    t = jnp.where(cnt >= _K, jnp.int32(0), jnp.int32(-2147483648))
    for bit in range(30, -1, -1):
        cand = t | (jnp.int32(1) << bit)
        cnt = jnp.sum((key >= cand).astype(jnp.int32), axis=1, keepdims=True)
        t = jnp.where(cnt >= _K, cand, t)
    return t


def _tc_kernel_body(x_ref, o_ref):
    x = x_ref[...]
    b = jax.lax.bitcast_convert_type(x, jnp.int32)
    key = b ^ ((b >> 31) & jnp.int32(0x7FFFFFFF))

    half = x.shape[0] // 2
    t0 = _search(key[:half])
    t1 = _search(key[half:])
    t = jnp.concatenate([t0, t1], axis=0)

    thr_bits = t ^ ((t >> 31) & jnp.int32(0x7FFFFFFF))
    thr = jax.lax.bitcast_convert_type(thr_bits, jnp.float32)
    o_ref[...] = jnp.where(x >= thr, x, jnp.zeros_like(x))


def _tc_call(x):
    n_tc = _ROWS - _SC_ROWS
    return pl.pallas_call(
        _tc_kernel_body,
        grid=(1,),
        in_specs=[pl.BlockSpec((n_tc, _COLS), lambda i: (1, 0))],
        out_specs=pl.BlockSpec((n_tc, _COLS), lambda i: (0, 0)),
        out_shape=jax.ShapeDtypeStruct((n_tc, _COLS), x.dtype),
    )(x)


@jax.jit
def kernel(x):
    out_sc = _sc_call(x)
    out_tc = _tc_call(x)
    return jnp.concatenate([out_sc, out_tc], axis=0)
